# baseline (speedup 1.0000x reference)
import jax
import jax.numpy as jnp
from jax import lax
from jax.experimental import pallas as pl
from jax.experimental.pallas import tpu as pltpu
from jax.experimental.pallas import tpu_sc as plsc

N_NODES = 10000
N_EDGES = 320000
D = 128
LANES = 16
N_FB = D // LANES
N_RANGES = 4
NODES_PER_RANGE = N_NODES // N_RANGES
CHUNK = 6400
N_CHUNKS = N_EDGES // CHUNK
GROUPS = CHUNK // LANES
PEND = CHUNK + 4 * LANES
NBUF = 8

_INF = float("inf")

N_CORES = 2
N_SUBCORES = 16


def _sc_body(src_hbm, dst_hbm, out_hbm,
             idx_a, idx_b, pend_ids, pend_loc, rows_v, acc_v, cfl_v, cfl2_v,
             sem_ia, sem_ib, *row_sems):
    wid = lax.axis_index("s") * N_CORES + lax.axis_index("c")
    fb = wid % N_FB
    nr = wid // N_FB
    base = nr * NODES_PER_RANGE

    iota = lax.iota(jnp.int32, LANES)
    iota8 = iota * N_FB
    f16s = [jnp.full((LANES,), f, jnp.int32) for f in range(LANES)]
    r16s = [iota + b * LANES for b in range(2 * NBUF)]

    def init_body(i, _):
        acc_v[i] = jnp.full((LANES,), _INF, jnp.float32)
        return 0
    lax.fori_loop(0, NODES_PER_RANGE + 1, init_body, 0)

    def pinit_body(i, _):
        pend_ids[pl.ds(i * LANES, LANES)] = jnp.zeros((LANES,), jnp.int32)
        return 0
    lax.fori_loop(0, PEND // LANES, pinit_body, 0)

    idx_refs = (idx_a, idx_b)
    idx_sems = (sem_ia, sem_ib)

    pltpu.async_copy(dst_hbm.at[pl.ds(0, CHUNK)], idx_a, sem_ia)
    pltpu.async_copy(dst_hbm.at[pl.ds(CHUNK, CHUNK)], idx_b, sem_ib)

    def update_block(blk, b):
        cfl = cfl_v if b % 2 == 0 else cfl2_v
        loc16 = pend_loc[pl.ds(blk * LANES, LANES)]
        plsc.store_scatter(cfl, [loc16], iota)
        rb = plsc.load_gather(cfl, [loc16])
        win16 = rb == iota
        nloss = plsc.all_reduce_population_count(rb != iota)[0]

        svs = [plsc.load_gather(rows_v, [r16s[b], f16s[f]])
               for f in range(LANES)]
        avs = [plsc.load_gather(acc_v, [loc16, f16s[f]])
               for f in range(LANES)]
        for f in range(LANES):
            plsc.store_scatter(acc_v, [loc16, f16s[f]],
                               jnp.minimum(avs[f], svs[f]), mask=win16)

        @pl.when(nloss > 0)
        def _slow():
            for l in range(LANES):
                li = loc16[l]
                acc_v[li] = jnp.minimum(acc_v[li], rows_v[b * LANES + l])

    def fire_slot(slot, b):
        pltpu.async_copy(src_hbm.at[pend_ids.at[pl.ds(slot * 2 * LANES,
                                                      2 * LANES)]],
                         rows_v.at[pl.ds(b * 2 * LANES, 2 * LANES)],
                         row_sems[b])

    def wait_slot(b):
        pltpu.make_async_copy(
            src_hbm.at[pl.ds(0, 2 * LANES)],
            rows_v.at[pl.ds(b * 2 * LANES, 2 * LANES)], row_sems[b]).wait()

    def do_chunk(c, parity, np_s):
        idx_v = idx_refs[parity]
        sem_i = idx_sems[parity]
        e0 = c * CHUNK
        pltpu.make_async_copy(dst_hbm.at[pl.ds(0, CHUNK)], idx_v,
                              sem_i).wait()

        def scan_grp(gg, np_vec):
            for u in range(4):
                g = gg * 4 + u
                d16 = idx_v[pl.ds(g * LANES, LANES)]
                loc16 = d16 - base
                ok16 = (loc16 >= 0) & (loc16 < NODES_PER_RANGE)
                prefix = plsc.cumsum(jnp.where(ok16, 1, 0))
                pos16 = np_vec + prefix - 1
                ids16 = iota8 + ((e0 + g * LANES) * N_FB + fb)
                plsc.store_scatter(pend_ids, [pos16], ids16, mask=ok16)
                plsc.store_scatter(pend_loc, [pos16], loc16, mask=ok16)
                np_vec = np_vec + plsc.all_reduce_population_count(ok16)
            return np_vec

        np_vec = lax.fori_loop(0, GROUPS // 4, scan_grp,
                               jnp.full((LANES,), np_s, jnp.int32))
        np_s = np_vec[0]

        @pl.when(c + 2 < N_CHUNKS)
        def _pf():
            pltpu.async_copy(dst_hbm.at[pl.ds((c + 2) * CHUNK, CHUNK)],
                             idx_v, sem_i)

        nfull = np_s // LANES

        nslot = (nfull + 1) // 2
        for b in range(NBUF):
            @pl.when(b < nslot)
            def _prime(b=b):
                fire_slot(b, b)

        nsuper = (nslot + NBUF - 1) // NBUF

        def super_body(sb, _):
            for b in range(NBUF):
                slot = sb * NBUF + b

                @pl.when(slot < nslot)
                def _one(slot=slot, b=b):
                    wait_slot(b)
                    update_block(2 * slot, 2 * b)

                    @pl.when(2 * slot + 1 < nfull)
                    def _second():
                        update_block(2 * slot + 1, 2 * b + 1)

                    @pl.when(slot + NBUF < nslot)
                    def _next():
                        fire_slot(slot + NBUF, b)
            return 0

        lax.fori_loop(0, nsuper, super_body, 0)

        rem = np_s - nfull * LANES
        t_ids = pend_ids[pl.ds(nfull * LANES, LANES)]
        t_loc = pend_loc[pl.ds(nfull * LANES, LANES)]
        pend_ids[pl.ds(0, LANES)] = t_ids
        pend_loc[pl.ds(0, LANES)] = t_loc
        return rem

    def pair_body(i, np_s):
        np_s = do_chunk(2 * i, 0, np_s)
        np_s = do_chunk(2 * i + 1, 1, np_s)
        return np_s

    np_s = lax.fori_loop(0, N_CHUNKS // 2, pair_body, jnp.int32(0))

    sel = iota < np_s
    ids16 = jnp.where(sel, pend_ids[pl.ds(0, LANES)], 0)
    loc16 = jnp.where(sel, pend_loc[pl.ds(0, LANES)], NODES_PER_RANGE)
    pltpu.async_copy(src_hbm.at[ids16], rows_v.at[pl.ds(0, LANES)],
                     row_sems[0])
    pltpu.make_async_copy(
        src_hbm.at[pl.ds(0, LANES)],
        rows_v.at[pl.ds(0, LANES)], row_sems[0]).wait()
    for l in range(LANES):
        li = loc16[l]
        acc_v[li] = jnp.minimum(acc_v[li], rows_v[l])

    def fin_body(i, _):
        v = acc_v[i]
        acc_v[i] = jnp.where(v == _INF, jnp.float32(0.0), v)
        return 0
    lax.fori_loop(0, NODES_PER_RANGE, fin_body, 0)

    pltpu.sync_copy(acc_v.at[pl.ds(0, NODES_PER_RANGE)],
                    out_hbm.at[pl.ds(base, NODES_PER_RANGE),
                               pl.ds(fb * LANES, LANES)])


@jax.jit
def _segment_min_sc(src16, dst):
    mesh = plsc.VectorSubcoreMesh(
        core_axis_name="c", subcore_axis_name="s",
        num_cores=N_CORES, num_subcores=N_SUBCORES)
    return pl.kernel(
        _sc_body,
        out_type=jax.ShapeDtypeStruct((N_NODES, D), jnp.float32),
        mesh=mesh,
        scratch_types=[
            pltpu.VMEM((CHUNK,), jnp.int32),
            pltpu.VMEM((CHUNK,), jnp.int32),
            pltpu.VMEM((PEND,), jnp.int32),
            pltpu.VMEM((PEND,), jnp.int32),
            pltpu.VMEM((2 * NBUF * LANES, LANES), jnp.float32),
            pltpu.VMEM((NODES_PER_RANGE + 1, LANES), jnp.float32),
            pltpu.VMEM((NODES_PER_RANGE + 1,), jnp.int32),
            pltpu.VMEM((NODES_PER_RANGE + 1,), jnp.int32),
            pltpu.SemaphoreType.DMA,
            pltpu.SemaphoreType.DMA,
        ] + [pltpu.SemaphoreType.DMA] * NBUF,
        compiler_params=pltpu.CompilerParams(
            use_tc_tiling_on_sc=False, needs_layout_passes=False),
    )(src16, dst)


def kernel(source_node_representation_with_coefficient, edge_index, feature_dim):
    src16 = source_node_representation_with_coefficient.reshape(
        N_EDGES * N_FB, LANES)
    dst = edge_index[1]
    return _segment_min_sc(src16, dst)


# speedup vs baseline: 1.2728x; 1.2728x over previous
import jax
import jax.numpy as jnp
from jax import lax
from jax.experimental import pallas as pl
from jax.experimental.pallas import tpu as pltpu
from jax.experimental.pallas import tpu_sc as plsc

N_NODES = 10000
N_EDGES = 320000
D = 128
LANES = 16
N_FB = D // LANES
N_RANGES = 4
NODES_PER_RANGE = N_NODES // N_RANGES
CHUNK = 6400
N_CHUNKS = N_EDGES // CHUNK
GROUPS = CHUNK // LANES
PEND = CHUNK + 4 * LANES
NBUF = 4

_INF = float("inf")

N_CORES = 2
N_SUBCORES = 16


def _sc_body(src_hbm, dst_hbm, out_hbm,
             idx_a, idx_b, pend_ids, pend_loc, rows_v, acc_v, cfl_v, cfl2_v,
             sem_ia, sem_ib, *row_sems):
    wid = lax.axis_index("s") * N_CORES + lax.axis_index("c")
    fb = wid % N_FB
    nr = wid // N_FB
    base = nr * NODES_PER_RANGE

    iota = lax.iota(jnp.int32, LANES)
    iota8 = iota * N_FB
    f16s = [jnp.full((LANES,), f, jnp.int32) for f in range(LANES)]
    r16s = [iota + b * LANES for b in range(NBUF)]

    def init_body(i, _):
        acc_v[i] = jnp.full((LANES,), _INF, jnp.float32)
        return 0
    lax.fori_loop(0, NODES_PER_RANGE + 1, init_body, 0)

    idx_refs = (idx_a, idx_b)
    idx_sems = (sem_ia, sem_ib)

    pltpu.async_copy(dst_hbm.at[pl.ds(0, CHUNK)], idx_a, sem_ia)
    pltpu.async_copy(dst_hbm.at[pl.ds(CHUNK, CHUNK)], idx_b, sem_ib)

    def update_block(blk, b):
        cfl = cfl_v if b % 2 == 0 else cfl2_v
        loc16 = pend_loc[pl.ds(blk * LANES, LANES)]
        plsc.store_scatter(cfl, [loc16], iota)
        rb = plsc.load_gather(cfl, [loc16])
        win16 = rb == iota
        nloss = plsc.all_reduce_population_count(rb != iota)[0]

        svs = [plsc.load_gather(rows_v, [r16s[b], f16s[f]])
               for f in range(LANES)]
        avs = [plsc.load_gather(acc_v, [loc16, f16s[f]])
               for f in range(LANES)]
        for f in range(LANES):
            plsc.store_scatter(acc_v, [loc16, f16s[f]],
                               jnp.minimum(avs[f], svs[f]), mask=win16)

        @pl.when(nloss > 0)
        def _slow():
            for l in range(LANES):
                li = loc16[l]
                acc_v[li] = jnp.minimum(acc_v[li], rows_v[b * LANES + l])

    def fire_block(blk, b):
        ids16 = pend_ids[pl.ds(blk * LANES, LANES)]
        pltpu.async_copy(src_hbm.at[ids16],
                         rows_v.at[pl.ds(b * LANES, LANES)], row_sems[b])

    def wait_block(b):
        pltpu.make_async_copy(
            src_hbm.at[pl.ds(0, LANES)],
            rows_v.at[pl.ds(b * LANES, LANES)], row_sems[b]).wait()

    def do_chunk(c, parity, np_s):
        idx_v = idx_refs[parity]
        sem_i = idx_sems[parity]
        e0 = c * CHUNK
        pltpu.make_async_copy(dst_hbm.at[pl.ds(0, CHUNK)], idx_v,
                              sem_i).wait()

        def scan_grp(gg, np_vec):
            for u in range(4):
                g = gg * 4 + u
                d16 = idx_v[pl.ds(g * LANES, LANES)]
                loc16 = d16 - base
                ok16 = (loc16 >= 0) & (loc16 < NODES_PER_RANGE)
                prefix = plsc.cumsum(jnp.where(ok16, 1, 0))
                pos16 = np_vec + prefix - 1
                ids16 = iota8 + ((e0 + g * LANES) * N_FB + fb)
                plsc.store_scatter(pend_ids, [pos16], ids16, mask=ok16)
                plsc.store_scatter(pend_loc, [pos16], loc16, mask=ok16)
                np_vec = np_vec + plsc.all_reduce_population_count(ok16)
            return np_vec

        np_vec = lax.fori_loop(0, GROUPS // 4, scan_grp,
                               jnp.full((LANES,), np_s, jnp.int32))
        np_s = np_vec[0]

        @pl.when(c + 2 < N_CHUNKS)
        def _pf():
            pltpu.async_copy(dst_hbm.at[pl.ds((c + 2) * CHUNK, CHUNK)],
                             idx_v, sem_i)

        nfull = np_s // LANES

        for b in range(NBUF):
            @pl.when(b < nfull)
            def _prime(b=b):
                fire_block(b, b)

        nsuper = (nfull + NBUF - 1) // NBUF

        def super_body(sb, _):
            for b in range(NBUF):
                blk = sb * NBUF + b

                @pl.when(blk < nfull)
                def _one(blk=blk, b=b):
                    wait_block(b)
                    update_block(blk, b)

                    @pl.when(blk + NBUF < nfull)
                    def _next():
                        fire_block(blk + NBUF, b)
            return 0

        lax.fori_loop(0, nsuper, super_body, 0)

        rem = np_s - nfull * LANES
        t_ids = pend_ids[pl.ds(nfull * LANES, LANES)]
        t_loc = pend_loc[pl.ds(nfull * LANES, LANES)]
        pend_ids[pl.ds(0, LANES)] = t_ids
        pend_loc[pl.ds(0, LANES)] = t_loc
        return rem

    def pair_body(i, np_s):
        np_s = do_chunk(2 * i, 0, np_s)
        np_s = do_chunk(2 * i + 1, 1, np_s)
        return np_s

    np_s = lax.fori_loop(0, N_CHUNKS // 2, pair_body, jnp.int32(0))

    sel = iota < np_s
    ids16 = jnp.where(sel, pend_ids[pl.ds(0, LANES)], 0)
    loc16 = jnp.where(sel, pend_loc[pl.ds(0, LANES)], NODES_PER_RANGE)
    pltpu.async_copy(src_hbm.at[ids16], rows_v.at[pl.ds(0, LANES)],
                     row_sems[0])
    wait_block(0)
    for l in range(LANES):
        li = loc16[l]
        acc_v[li] = jnp.minimum(acc_v[li], rows_v[l])

    def fin_body(i, _):
        v = acc_v[i]
        acc_v[i] = jnp.where(v == _INF, jnp.float32(0.0), v)
        return 0
    lax.fori_loop(0, NODES_PER_RANGE, fin_body, 0)

    pltpu.sync_copy(acc_v.at[pl.ds(0, NODES_PER_RANGE)],
                    out_hbm.at[pl.ds(base, NODES_PER_RANGE),
                               pl.ds(fb * LANES, LANES)])


@jax.jit
def _segment_min_sc(src16, dst):
    mesh = plsc.VectorSubcoreMesh(
        core_axis_name="c", subcore_axis_name="s",
        num_cores=N_CORES, num_subcores=N_SUBCORES)
    return pl.kernel(
        _sc_body,
        out_type=jax.ShapeDtypeStruct((N_NODES, D), jnp.float32),
        mesh=mesh,
        scratch_types=[
            pltpu.VMEM((CHUNK,), jnp.int32),
            pltpu.VMEM((CHUNK,), jnp.int32),
            pltpu.VMEM((PEND,), jnp.int32),
            pltpu.VMEM((PEND,), jnp.int32),
            pltpu.VMEM((NBUF * LANES, LANES), jnp.float32),
            pltpu.VMEM((NODES_PER_RANGE + 1, LANES), jnp.float32),
            pltpu.VMEM((NODES_PER_RANGE + 1,), jnp.int32),
            pltpu.VMEM((NODES_PER_RANGE + 1,), jnp.int32),
            pltpu.SemaphoreType.DMA,
            pltpu.SemaphoreType.DMA,
        ] + [pltpu.SemaphoreType.DMA] * NBUF,
        compiler_params=pltpu.CompilerParams(
            use_tc_tiling_on_sc=False, needs_layout_passes=False),
    )(src16, dst)


def kernel(source_node_representation_with_coefficient, edge_index, feature_dim):
    src16 = source_node_representation_with_coefficient.reshape(
        N_EDGES * N_FB, LANES)
    dst = edge_index[1]
    return _segment_min_sc(src16, dst)


# speedup vs baseline: 1.5103x; 1.1866x over previous
import jax
import jax.numpy as jnp
from jax import lax
from jax.experimental import pallas as pl
from jax.experimental.pallas import tpu as pltpu
from jax.experimental.pallas import tpu_sc as plsc

N_NODES = 10000
N_EDGES = 320000
D = 128
LANES = 16
N_FB = D // LANES
N_RANGES = 4
NODES_PER_RANGE = N_NODES // N_RANGES
CHUNK = 6400
N_CHUNKS = N_EDGES // CHUNK
GROUPS = CHUNK // LANES
PEND = CHUNK + 4 * LANES
NBUF = 8

_INF = float("inf")

N_CORES = 2
N_SUBCORES = 16


def _sc_body(src_hbm, dst_hbm, out_hbm,
             idx_a, idx_b, pend_ids, pend_loc, rows_v, acc_v, cfl_v, cfl2_v,
             sem_ia, sem_ib, *row_sems):
    wid = lax.axis_index("s") * N_CORES + lax.axis_index("c")
    fb = wid % N_FB
    nr = wid // N_FB
    base = nr * NODES_PER_RANGE

    iota = lax.iota(jnp.int32, LANES)
    iota8 = iota * N_FB
    f16s = [jnp.full((LANES,), f, jnp.int32) for f in range(LANES)]
    r16s = [iota + b * LANES for b in range(NBUF)]

    def init_body(i, _):
        acc_v[i] = jnp.full((LANES,), _INF, jnp.float32)
        return 0
    lax.fori_loop(0, NODES_PER_RANGE + 1, init_body, 0)

    idx_refs = (idx_a, idx_b)
    idx_sems = (sem_ia, sem_ib)

    pltpu.async_copy(dst_hbm.at[pl.ds(0, CHUNK)], idx_a, sem_ia)
    pltpu.async_copy(dst_hbm.at[pl.ds(CHUNK, CHUNK)], idx_b, sem_ib)

    def update_block(blk, b):
        cfl = cfl_v if b % 2 == 0 else cfl2_v
        loc16 = pend_loc[pl.ds(blk * LANES, LANES)]
        plsc.store_scatter(cfl, [loc16], iota)
        rb = plsc.load_gather(cfl, [loc16])
        win16 = rb == iota
        nloss = plsc.all_reduce_population_count(rb != iota)[0]

        svs = [plsc.load_gather(rows_v, [r16s[b], f16s[f]])
               for f in range(LANES)]
        avs = [plsc.load_gather(acc_v, [loc16, f16s[f]])
               for f in range(LANES)]
        for f in range(LANES):
            plsc.store_scatter(acc_v, [loc16, f16s[f]],
                               jnp.minimum(avs[f], svs[f]), mask=win16)

        @pl.when(nloss > 0)
        def _slow():
            for l in range(LANES):
                li = loc16[l]
                acc_v[li] = jnp.minimum(acc_v[li], rows_v[b * LANES + l])

    def fire_block(blk, b):
        ids16 = pend_ids[pl.ds(blk * LANES, LANES)]
        pltpu.async_copy(src_hbm.at[ids16],
                         rows_v.at[pl.ds(b * LANES, LANES)], row_sems[b])

    def wait_block(b):
        pltpu.make_async_copy(
            src_hbm.at[pl.ds(0, LANES)],
            rows_v.at[pl.ds(b * LANES, LANES)], row_sems[b]).wait()

    def do_chunk(c, parity, np_s):
        idx_v = idx_refs[parity]
        sem_i = idx_sems[parity]
        e0 = c * CHUNK
        pltpu.make_async_copy(dst_hbm.at[pl.ds(0, CHUNK)], idx_v,
                              sem_i).wait()

        def scan_grp(gg, np_vec):
            for u in range(4):
                g = gg * 4 + u
                d16 = idx_v[pl.ds(g * LANES, LANES)]
                loc16 = d16 - base
                ok16 = (loc16 >= 0) & (loc16 < NODES_PER_RANGE)
                prefix = plsc.cumsum(jnp.where(ok16, 1, 0))
                pos16 = np_vec + prefix - 1
                ids16 = iota8 + ((e0 + g * LANES) * N_FB + fb)
                plsc.store_scatter(pend_ids, [pos16], ids16, mask=ok16)
                plsc.store_scatter(pend_loc, [pos16], loc16, mask=ok16)
                np_vec = np_vec + plsc.all_reduce_population_count(ok16)
            return np_vec

        np_vec = lax.fori_loop(0, GROUPS // 4, scan_grp,
                               jnp.full((LANES,), np_s, jnp.int32))
        np_s = np_vec[0]

        @pl.when(c + 2 < N_CHUNKS)
        def _pf():
            pltpu.async_copy(dst_hbm.at[pl.ds((c + 2) * CHUNK, CHUNK)],
                             idx_v, sem_i)

        nfull = np_s // LANES

        for b in range(NBUF):
            @pl.when(b < nfull)
            def _prime(b=b):
                fire_block(b, b)

        nsuper = (nfull + NBUF - 1) // NBUF

        def super_body(sb, _):
            for b in range(NBUF):
                blk = sb * NBUF + b

                @pl.when(blk < nfull)
                def _one(blk=blk, b=b):
                    wait_block(b)
                    update_block(blk, b)

                    @pl.when(blk + NBUF < nfull)
                    def _next():
                        fire_block(blk + NBUF, b)
            return 0

        lax.fori_loop(0, nsuper, super_body, 0)

        rem = np_s - nfull * LANES
        t_ids = pend_ids[pl.ds(nfull * LANES, LANES)]
        t_loc = pend_loc[pl.ds(nfull * LANES, LANES)]
        pend_ids[pl.ds(0, LANES)] = t_ids
        pend_loc[pl.ds(0, LANES)] = t_loc
        return rem

    def pair_body(i, np_s):
        np_s = do_chunk(2 * i, 0, np_s)
        np_s = do_chunk(2 * i + 1, 1, np_s)
        return np_s

    np_s = lax.fori_loop(0, N_CHUNKS // 2, pair_body, jnp.int32(0))

    sel = iota < np_s
    ids16 = jnp.where(sel, pend_ids[pl.ds(0, LANES)], 0)
    loc16 = jnp.where(sel, pend_loc[pl.ds(0, LANES)], NODES_PER_RANGE)
    pltpu.async_copy(src_hbm.at[ids16], rows_v.at[pl.ds(0, LANES)],
                     row_sems[0])
    wait_block(0)
    for l in range(LANES):
        li = loc16[l]
        acc_v[li] = jnp.minimum(acc_v[li], rows_v[l])

    def fin_body(i, _):
        v = acc_v[i]
        acc_v[i] = jnp.where(v == _INF, jnp.float32(0.0), v)
        return 0
    lax.fori_loop(0, NODES_PER_RANGE, fin_body, 0)

    pltpu.sync_copy(acc_v.at[pl.ds(0, NODES_PER_RANGE)],
                    out_hbm.at[pl.ds(base, NODES_PER_RANGE),
                               pl.ds(fb * LANES, LANES)])


@jax.jit
def _segment_min_sc(src16, dst):
    mesh = plsc.VectorSubcoreMesh(
        core_axis_name="c", subcore_axis_name="s",
        num_cores=N_CORES, num_subcores=N_SUBCORES)
    return pl.kernel(
        _sc_body,
        out_type=jax.ShapeDtypeStruct((N_NODES, D), jnp.float32),
        mesh=mesh,
        scratch_types=[
            pltpu.VMEM((CHUNK,), jnp.int32),
            pltpu.VMEM((CHUNK,), jnp.int32),
            pltpu.VMEM((PEND,), jnp.int32),
            pltpu.VMEM((PEND,), jnp.int32),
            pltpu.VMEM((NBUF * LANES, LANES), jnp.float32),
            pltpu.VMEM((NODES_PER_RANGE + 1, LANES), jnp.float32),
            pltpu.VMEM((NODES_PER_RANGE + 1,), jnp.int32),
            pltpu.VMEM((NODES_PER_RANGE + 1,), jnp.int32),
            pltpu.SemaphoreType.DMA,
            pltpu.SemaphoreType.DMA,
        ] + [pltpu.SemaphoreType.DMA] * NBUF,
        compiler_params=pltpu.CompilerParams(
            use_tc_tiling_on_sc=False, needs_layout_passes=False),
    )(src16, dst)


def kernel(source_node_representation_with_coefficient, edge_index, feature_dim):
    src16 = source_node_representation_with_coefficient.reshape(
        N_EDGES * N_FB, LANES)
    dst = edge_index[1]
    return _segment_min_sc(src16, dst)
